# Initial kernel scaffold; baseline (speedup 1.0000x reference)
import jax
import jax.numpy as jnp
from jax.experimental import pallas as pl


def kernel(x, edge_index, edge_attr, eW1, eb1, eW2, eb2, cWs, cbs, cWm, cbm, hW1, hb1, hW2, hb2):
    raise NotImplementedError("write your pallas kernel here")


# speedup vs baseline: 3.4081x; 3.4081x over previous
import functools

import jax
import jax.numpy as jnp
from jax import lax
from jax.experimental import pallas as pl
from jax.experimental.pallas import tpu as pltpu
from jax.experimental.pallas import tpu_sc as plsc

NC = 2
NS = 16
NW = NC * NS
CH = 128
ZROWS = 64


def _sc_scatter_make(n_nodes, n_pad, feat, e_pad, gather):
  epw = e_pad // NW
  nchunk = epw // CH
  rpt = n_pad // NS
  pk = 128 // feat
  mesh = plsc.VectorSubcoreMesh(
      core_axis_name="c", subcore_axis_name="s", num_cores=NC,
      num_subcores=NS)

  scratch = [
      pltpu.VMEM((CH,), jnp.int32),
      pltpu.VMEM((CH,), jnp.int32),
      pltpu.VMEM((CH, 128), jnp.float32),
      pltpu.VMEM((CH // pk, 128), jnp.float32),
      pltpu.VMEM((ZROWS, 128), jnp.float32),
      pltpu.VMEM_SHARED((n_pad, 128), jnp.float32),
      pltpu.SemaphoreType.DMA,
  ]

  @functools.partial(
      pl.kernel,
      out_type=jax.ShapeDtypeStruct((NC * n_pad, 128), jnp.float32),
      mesh=mesh,
      scratch_types=scratch,
  )
  def k(tab_hbm, src_hbm, dst_hbm, out_hbm, src_v, dst_v, rows_v, pk_v,
        zb_v, acc_sh, sem):
    c = lax.axis_index("c")
    s = lax.axis_index("s")
    wid = s * NC + c

    def zb_body(i, _):
      zb_v[i // 8, pl.ds((i % 8) * 16, 16)] = jnp.zeros((16,), jnp.float32)
      return 0

    lax.fori_loop(0, ZROWS * 8, zb_body, 0)
    if not gather:
      def rz_body(i, _):
        rows_v[i // 8, pl.ds((i % 8) * 16, 16)] = jnp.zeros((16,),
                                                            jnp.float32)
        return 0

      lax.fori_loop(0, CH * 8, rz_body, 0)
    for i in range(rpt // ZROWS):
      pltpu.sync_copy(zb_v, acc_sh.at[pl.ds(s * rpt + i * ZROWS, ZROWS)])
    plsc.subcore_barrier()

    base = wid * epw

    def body(j, _):
      off = base + j * CH
      pltpu.sync_copy(dst_hbm.at[pl.ds(off, CH)], dst_v)
      if gather:
        pltpu.sync_copy(src_hbm.at[pl.ds(off, CH)], src_v)
        pltpu.async_copy(tab_hbm.at[src_v], rows_v, sem).wait()
      else:
        pltpu.sync_copy(
            tab_hbm.at[pl.ds(pl.multiple_of(off // pk, 8), CH // pk)],
            pk_v)

        def exp_body(r, _):
          for v in range(feat // 16):
            rows_v[r, pl.ds(v * 16, 16)] = pk_v[r // pk,
                                                pl.ds((r % pk) * feat
                                                      + v * 16, 16)]
          return 0

        lax.fori_loop(0, CH, exp_body, 0)
      pltpu.sync_copy(rows_v, acc_sh.at[dst_v], add=True)
      return 0

    lax.fori_loop(0, nchunk, body, 0)
    plsc.subcore_barrier()

    for i in range(rpt // CH):
      r0 = s * rpt + i * CH
      pltpu.sync_copy(acc_sh.at[pl.ds(r0, CH)], rows_v)
      pltpu.sync_copy(rows_v, out_hbm.at[pl.ds(c * n_pad + r0, CH)])

  return k


def _dot(a, b):
  return jnp.dot(a, b, preferred_element_type=jnp.float32,
                 precision=lax.Precision.HIGHEST)


def _embed_body(x_ref, w1, b1, w2, b2, wm, bm, h_ref, hw_ref):
  t = jnp.maximum(_dot(x_ref[...], w1[...]) + b1[...], 0.0)
  h = _dot(t, w2[...]) + b2[...]
  h_ref[...] = h
  hw_ref[...] = _dot(h, wm[...]) + bm[...]


def _update_body(h_ref, a0, a1, e0, e1, wme, ws, bs, wm, bm, h2_ref,
                 hw2_ref):
  agg = a0[...] + a1[...] + _dot(e0[...] + e1[...], wme[...])
  h2 = jnp.maximum(_dot(h_ref[...], ws[...]) + bs[...] + agg, 0.0)
  h2_ref[...] = h2
  hw2_ref[...] = _dot(h2, wm[...]) + bm[...]


def _final_body(h_ref, a0, a1, e0, e1, wme, ws, bs, w1, b1, w2, b2,
                o_ref):
  agg = a0[...] + a1[...] + _dot(e0[...] + e1[...], wme[...])
  h2 = jnp.maximum(_dot(h_ref[...], ws[...]) + bs[...] + agg, 0.0)
  t = jnp.maximum(_dot(h2, w1[...]) + b1[...], 0.0)
  o_ref[...] = _dot(t, w2[...]) + b2[...]


def _f32(*shapes):
  return tuple(jax.ShapeDtypeStruct(s, jnp.float32) for s in shapes)


def kernel(x, edge_index, edge_attr, eW1, eb1, eW2, eb2, cWs, cbs, cWm,
           cbm, hW1, hb1, hW2, hb2):
  n, h_dim = x.shape
  e = edge_index.shape[1]
  ed = edge_attr.shape[1]
  l_layers = cWs.shape[0]
  out_dim = hW2.shape[1]

  n_pad = ((n + NS * ZROWS) // (NS * ZROWS)) * (NS * ZROWS)
  e_pad = ((e + NW * CH - 1) // (NW * CH)) * (NW * CH)

  src = edge_index[0].astype(jnp.int32)
  dst = edge_index[1].astype(jnp.int32)
  src_p = jnp.concatenate([src, jnp.zeros((e_pad - e,), jnp.int32)])
  dst_p = jnp.concatenate(
      [dst, jnp.full((e_pad - e,), n, jnp.int32)])
  ea_p = jnp.concatenate(
      [edge_attr, jnp.zeros((e_pad - e, ed), jnp.float32)])

  wmh = cWm[:, :h_dim, :]
  wme = cWm[:, h_dim:, :]
  b = lambda v: v.reshape(1, -1)

  sc_edge = _sc_scatter_make(n, n_pad, h_dim, e_pad, gather=True)
  sc_ea = _sc_scatter_make(n, n_pad, ed, e_pad, gather=False)

  halves = lambda a: (a[:n], a[n_pad:n_pad + n])

  ea8 = ea_p.reshape(e_pad // (128 // ed), 128)
  ea_out = sc_ea(ea8, src_p, dst_p)
  ea0, ea1 = ea_out[:n, :ed], ea_out[n_pad:n_pad + n, :ed]

  h, hw = pl.pallas_call(
      _embed_body, out_shape=_f32((n, h_dim), (n, h_dim)))(
          x, eW1, b(eb1), eW2, b(eb2), wmh[0], b(cbm[0]))

  for l in range(l_layers - 1):
    a0, a1 = halves(sc_edge(hw, src_p, dst_p))
    h, hw = pl.pallas_call(
        _update_body, out_shape=_f32((n, h_dim), (n, h_dim)))(
            h, a0, a1, ea0, ea1, wme[l], cWs[l], b(cbs[l]),
            wmh[l + 1], b(cbm[l + 1]))

  a0, a1 = halves(sc_edge(hw, src_p, dst_p))
  w2p = jnp.zeros((h_dim, 128), jnp.float32).at[:, :out_dim].set(hW2)
  b2p = jnp.zeros((1, 128), jnp.float32).at[0, :out_dim].set(hb2)
  out = pl.pallas_call(
      _final_body, out_shape=jax.ShapeDtypeStruct((n, 128), jnp.float32))(
          h, a0, a1, ea0, ea1, wme[l_layers - 1],
          cWs[l_layers - 1], b(cbs[l_layers - 1]), hW1, b(hb1), w2p, b2p)
  return out[:, :out_dim]
